# Initial kernel scaffold; baseline (speedup 1.0000x reference)
import jax
import jax.numpy as jnp
from jax.experimental import pallas as pl


def kernel(x, edge_index, edge_attr, batch, params):
    raise NotImplementedError("write your pallas kernel here")


# speedup vs baseline: 1.9353x; 1.9353x over previous
import functools

import jax
import jax.numpy as jnp
from jax import lax
from jax.experimental import pallas as pl
from jax.experimental.pallas import tpu as pltpu
from jax.experimental.pallas import tpu_sc as plsc

EMB = 128
NTASK = 10
NGRAPH = 128
GAMMA = 0.4

NC = 2
NS = 16
NW = NC * NS

CHUNK = 128

N_PALLAS_LAYERS = 2


@functools.lru_cache(maxsize=None)
def _make_conv_sc(nchunk: int, agg_rows: int):
    rows_per_tile = agg_rows // NS
    mesh = plsc.VectorSubcoreMesh(core_axis_name="c", subcore_axis_name="s")

    def body(h_hbm, srcs_hbm, dsts_hbm, codes_hbm, tab_hbm, zeros_hbm,
             out_hbm, idx_s, idx_d, idx_c, buf_h, buf_e, agg_sh,
             sem_h, sem_e):
        c = lax.axis_index("c")
        s = lax.axis_index("s")
        wid = s * NC + c

        pltpu.sync_copy(zeros_hbm.at[pl.ds(s * rows_per_tile, rows_per_tile)],
                        agg_sh.at[pl.ds(s * rows_per_tile, rows_per_tile)])
        plsc.subcore_barrier()

        def chunk_body(j, carry):
            pltpu.sync_copy(srcs_hbm.at[wid, j], idx_s)
            pltpu.sync_copy(dsts_hbm.at[wid, j], idx_d)
            pltpu.sync_copy(codes_hbm.at[wid, j], idx_c)
            pltpu.async_copy(h_hbm.at[idx_s], buf_h, sem_h).wait()
            pltpu.async_copy(tab_hbm.at[idx_c], buf_e, sem_e).wait()

            def edge_body(i, cc):
                for jb in range(EMB // 16):
                    sl = pl.ds(jb * 16, 16)
                    buf_h[i, sl] = jnp.maximum(buf_h[i, sl] + buf_e[i, sl],
                                               0.0)
                return cc

            lax.fori_loop(0, CHUNK, edge_body, 0)
            pltpu.sync_copy(buf_h, agg_sh.at[idx_d], add=True)
            return carry

        lax.fori_loop(0, nchunk, chunk_body, 0)

        plsc.subcore_barrier()
        pltpu.sync_copy(agg_sh.at[pl.ds(s * rows_per_tile, rows_per_tile)],
                        out_hbm.at[c, pl.ds(s * rows_per_tile, rows_per_tile)])

    return pl.kernel(
        body,
        out_type=jax.ShapeDtypeStruct((NC, agg_rows, EMB), jnp.float32),
        mesh=mesh,
        scratch_types=[
            pltpu.VMEM((CHUNK,), jnp.int32),
            pltpu.VMEM((CHUNK,), jnp.int32),
            pltpu.VMEM((CHUNK,), jnp.int32),
            pltpu.VMEM((CHUNK, EMB), jnp.float32),
            pltpu.VMEM((CHUNK, EMB), jnp.float32),
            pltpu.VMEM_SHARED((agg_rows, EMB), jnp.float32),
            pltpu.SemaphoreType.DMA,
            pltpu.SemaphoreType.DMA,
        ],
    )


def _mm(x, w):
    return jnp.dot(x, w, preferred_element_type=jnp.float32)


def _bn(x, g, be):
    m = jnp.mean(x, axis=0, keepdims=True)
    v = jnp.mean((x - m) ** 2, axis=0, keepdims=True)
    return g * (x - m) / jnp.sqrt(v + 1e-5) + be


def _dense_body(relu_out, n_nodes, h_ref, agg_ref, eps_ref, w1_ref, b1_ref,
                g1_ref, be1_ref, w2_ref, b2_ref, g2_ref, be2_ref, out_ref):
    h = h_ref[...]
    agg = agg_ref[0, :n_nodes, :] + agg_ref[1, :n_nodes, :]
    z = (1.0 + eps_ref[0, 0]) * h + agg
    a = _mm(z, w1_ref[...]) + b1_ref[...]
    a = jax.nn.relu(_bn(a, g1_ref[...], be1_ref[...]))
    o = _mm(a, w2_ref[...]) + b2_ref[...]
    o = _bn(o, g2_ref[...], be2_ref[...])
    if relu_out:
        o = jax.nn.relu(o)
    out_ref[...] = o + h


def _head_body(h_ref, gate_ref, batch_ref, pw1_ref, pb1_ref, pg_ref, pbe_ref,
               pw2_ref, pb2_ref, pred_ref, loss_ref):
    gate = gate_ref[...]
    n = gate.shape[0]
    oh = (batch_ref[...] ==
          lax.broadcasted_iota(jnp.int32, (n, NGRAPH), 1)).astype(jnp.float32)
    dn = (((0,), (0,)), ((), ()))
    hp = jax.lax.Precision.HIGHEST
    counts = lax.dot_general(oh, jnp.ones((n, 1), jnp.float32), dn,
                             precision=hp,
                             preferred_element_type=jnp.float32)
    counts = jnp.maximum(counts, 1.0)
    hsum = lax.dot_general(oh, gate * h_ref[...], dn, precision=hp,
                           preferred_element_type=jnp.float32)
    r_num = lax.dot_general(oh, gate, dn, precision=hp,
                            preferred_element_type=jnp.float32) + 1e-8
    e_num = lax.dot_general(oh, 1.0 - gate, dn, precision=hp,
                            preferred_element_type=jnp.float32) + 1e-8
    h_out = hsum / counts

    p = _mm(h_out, pw1_ref[...]) + pb1_ref[...]
    p = jax.nn.relu(_bn(p, pg_ref[...], pbe_ref[...]))
    pred_ref[...] = _mm(p, pw2_ref[...]) + pb2_ref[...]
    loss = jnp.mean(jnp.abs(r_num / (r_num + e_num) - GAMMA))
    loss_ref[...] = jnp.broadcast_to(loss, (1, 1))


def _tc_call(body, out_shape, *args):
    return pl.pallas_call(body, out_shape=out_shape)(*args)


def _edge_table(p):
    e = p["edge_emb"]
    t = (e[0][:, None, None, :] + e[1][None, :, None, :]
         + e[2][None, None, :, :])
    return t.reshape(4 * 4 * 4, EMB)


def _row(v):
    return v.reshape(1, -1)


def kernel(x, edge_index, edge_attr, batch, params):
    n = x.shape[0]
    n_edges = edge_index.shape[1]
    src = edge_index[0].astype(jnp.int32)
    dst = edge_index[1].astype(jnp.int32)
    ea = edge_attr.astype(jnp.int32)
    code = ea[:, 0] * 16 + ea[:, 1] * 4 + ea[:, 2]

    per_tile = -(-n_edges // (NW * CHUNK)) * CHUNK
    nchunk = per_tile // CHUNK
    pad = per_tile * NW - n_edges
    agg_rows = -(-n // (NS * 8)) * (NS * 8)
    if agg_rows == n:
        agg_rows += NS * 8

    src_p = jnp.concatenate([src, jnp.zeros((pad,), jnp.int32)]
                            ).reshape(NW, nchunk, CHUNK)
    dst_p = jnp.concatenate([dst, jnp.full((pad,), n, jnp.int32)]
                            ).reshape(NW, nchunk, CHUNK)
    code_p = jnp.concatenate([code, jnp.zeros((pad,), jnp.int32)]
                             ).reshape(NW, nchunk, CHUNK)
    zeros_hbm = jnp.zeros((agg_rows, EMB), jnp.float32)
    conv_sc = _make_conv_sc(nchunk, agg_rows)

    def _lin(p, t):
        return t @ p["W"] + p["b"]

    def _bnr(p, t):
        m = t.mean(0)
        v = t.var(0)
        return p["g"] * (t - m) / jnp.sqrt(v + 1e-5) + p["be"]

    def _gin_conv(p, h):
        ee = _edge_table(p)[code]
        msg = jax.nn.relu(h[src] + ee)
        agg = jax.ops.segment_sum(msg, dst, num_segments=n)
        z = (1.0 + p["eps"]) * h + agg
        z = _lin(p["mlp1"], z)
        z = _bnr(p["mlp_bn"], z)
        z = jax.nn.relu(z)
        z = _lin(p["mlp2"], z)
        return z

    def _layer_ref(p, h, relu_out):
        z = _bnr(p["bn"], _gin_conv(p, h))
        if relu_out:
            z = jax.nn.relu(z)
        return z + h

    def _layer_pallas(p, h, relu_out):
        agg2 = conv_sc(h, src_p, dst_p, code_p, _edge_table(p), zeros_hbm)
        dense = functools.partial(_dense_body, relu_out, n)
        return _tc_call(dense,
                        jax.ShapeDtypeStruct((n, EMB), jnp.float32),
                        h, agg2, p["eps"].reshape(1, 1),
                        p["mlp1"]["W"], _row(p["mlp1"]["b"]),
                        _row(p["mlp_bn"]["g"]), _row(p["mlp_bn"]["be"]),
                        p["mlp2"]["W"], _row(p["mlp2"]["b"]),
                        _row(p["bn"]["g"]), _row(p["bn"]["be"]))

    def _gnn(layers, h, n_pallas):
        k = len(layers)
        for i, p in enumerate(layers):
            fn = _layer_pallas if i >= k - n_pallas else _layer_ref
            h = fn(p, h, i < k - 1)
        return h

    xe = _lin(params["node_enc"], x)
    h_node = _gnn(params["gnn"], xe, N_PALLAS_LAYERS)
    x2 = _gnn(params["rat_gnn"], xe, 0)

    g = _lin(params["gate2"],
             jax.nn.relu(_bnr(params["gate_bn"], _lin(params["gate1"], x2))))
    gum = -jnp.log(-jnp.log(jax.random.uniform(
        jax.random.key(42), g.shape, jnp.float32, 1e-10, 1.0)))
    ysoft = jax.nn.softmax(g + gum, axis=-1)
    yhard = jax.nn.one_hot(jnp.argmax(ysoft, axis=-1), 2, dtype=jnp.float32)
    y = yhard + ysoft - jax.lax.stop_gradient(ysoft)
    gate = y[:, -1:]

    pred, loss = _tc_call(
        _head_body,
        (jax.ShapeDtypeStruct((NGRAPH, NTASK), jnp.float32),
         jax.ShapeDtypeStruct((1, 1), jnp.float32)),
        h_node, gate, batch.astype(jnp.int32).reshape(n, 1),
        params["pred1"]["W"], _row(params["pred1"]["b"]),
        _row(params["pred_bn"]["g"]), _row(params["pred_bn"]["be"]),
        params["pred2"]["W"], _row(params["pred2"]["b"]))

    return pred, loss.reshape(())
